# baseline (device time: 49049 ns/iter reference)
import jax
import jax.numpy as jnp
from jax import lax
from jax.experimental import pallas as pl
from jax.experimental.pallas import tpu as pltpu

N_DEV = 4


def kernel(x, W1, W2):
    m, k = x.shape
    _, h_per = W1.shape
    _, n = W2.shape

    def body(x_ref, w1_ref, w2_ref, out_ref, comm_ref, send_sems, recv_sems):
        my_pos = lax.axis_index("i")
        left = (my_pos - 1) % N_DEV
        right = (my_pos + 1) % N_DEV

        barrier_sem = pltpu.get_barrier_semaphore()
        for nbr in [left, right]:
            pl.semaphore_signal(
                barrier_sem, inc=1,
                device_id=(nbr,), device_id_type=pl.DeviceIdType.MESH,
            )
        pl.semaphore_wait(barrier_sem, 2)

        hid = jnp.maximum(
            jnp.dot(x_ref[:, :], w1_ref[:, :], preferred_element_type=jnp.float32),
            0.0,
        )
        partial = jnp.dot(hid, w2_ref[:, :], preferred_element_type=jnp.float32)
        out_ref[:, :] = partial
        comm_ref[0, :, :] = partial

        for h in range(N_DEV - 1):
            rdma = pltpu.make_async_remote_copy(
                src_ref=comm_ref.at[h],
                dst_ref=comm_ref.at[h + 1],
                send_sem=send_sems.at[h],
                recv_sem=recv_sems.at[h],
                device_id=(right,),
                device_id_type=pl.DeviceIdType.MESH,
            )
            rdma.start()
            rdma.wait()
            out_ref[:, :] += comm_ref[h + 1, :, :]

    return pl.pallas_call(
        body,
        out_shape=jax.ShapeDtypeStruct((m, n), jnp.float32),
        in_specs=[
            pl.BlockSpec(memory_space=pltpu.VMEM),
            pl.BlockSpec(memory_space=pltpu.VMEM),
            pl.BlockSpec(memory_space=pltpu.VMEM),
        ],
        out_specs=pl.BlockSpec(memory_space=pltpu.VMEM),
        scratch_shapes=[
            pltpu.VMEM((N_DEV, m, n), jnp.float32),
            pltpu.SemaphoreType.DMA((N_DEV - 1,)),
            pltpu.SemaphoreType.DMA((N_DEV - 1,)),
        ],
        compiler_params=pltpu.CompilerParams(collective_id=0),
    )(x, W1, W2)


# device time: 26313 ns/iter; 1.8641x vs baseline; 1.8641x over previous
import jax
import jax.numpy as jnp
from jax import lax
from jax.experimental import pallas as pl
from jax.experimental.pallas import tpu as pltpu

N_DEV = 4
NB = 4


def kernel(x, W1, W2):
    m, k = x.shape
    _, h_per = W1.shape
    _, n = W2.shape
    mb = m // NB

    def body(x_ref, w1_ref, w2_ref, out_ref,
             accA, accB, recvA, recvB,
             sendA_sems, recvA_sems, sendB_sems, recvB_sems):
        p = lax.axis_index("i")
        nbrA = p ^ 1
        nbrB = 3 - p

        barrier_sem = pltpu.get_barrier_semaphore()
        for nbr in [nbrA, nbrB]:
            pl.semaphore_signal(
                barrier_sem, inc=1,
                device_id=(nbr,), device_id_type=pl.DeviceIdType.MESH,
            )
        pl.semaphore_wait(barrier_sem, 2)

        rdmaA = [
            pltpu.make_async_remote_copy(
                src_ref=accA.at[b], dst_ref=recvA.at[b],
                send_sem=sendA_sems.at[b], recv_sem=recvA_sems.at[b],
                device_id=(nbrA,), device_id_type=pl.DeviceIdType.MESH,
            )
            for b in range(NB)
        ]
        rdmaB = [
            pltpu.make_async_remote_copy(
                src_ref=accB.at[b], dst_ref=recvB.at[b],
                send_sem=sendB_sems.at[b], recv_sem=recvB_sems.at[b],
                device_id=(nbrB,), device_id_type=pl.DeviceIdType.MESH,
            )
            for b in range(NB)
        ]

        for b in range(NB):
            xb = x_ref[b * mb:(b + 1) * mb, :]
            hb = jnp.maximum(
                jnp.dot(xb, w1_ref[:, :], preferred_element_type=jnp.float32),
                0.0,
            )
            accA[b] = jnp.dot(hb, w2_ref[:, :], preferred_element_type=jnp.float32)
            rdmaA[b].start()

        for b in range(NB):
            rdmaA[b].wait_recv()
            accB[b] = accA[b] + recvA[b]
            rdmaB[b].start()
            if b >= 1:
                rdmaB[b - 1].wait_recv()
                out_ref[(b - 1) * mb:b * mb, :] = accB[b - 1] + recvB[b - 1]
        rdmaB[NB - 1].wait_recv()
        out_ref[(NB - 1) * mb:, :] = accB[NB - 1] + recvB[NB - 1]

        for b in range(NB):
            rdmaA[b].wait_send()
            rdmaB[b].wait_send()

    return pl.pallas_call(
        body,
        out_shape=jax.ShapeDtypeStruct((m, n), jnp.float32),
        in_specs=[
            pl.BlockSpec(memory_space=pltpu.VMEM),
            pl.BlockSpec(memory_space=pltpu.VMEM),
            pl.BlockSpec(memory_space=pltpu.VMEM),
        ],
        out_specs=pl.BlockSpec(memory_space=pltpu.VMEM),
        scratch_shapes=[
            pltpu.VMEM((NB, mb, n), jnp.float32),
            pltpu.VMEM((NB, mb, n), jnp.float32),
            pltpu.VMEM((NB, mb, n), jnp.float32),
            pltpu.VMEM((NB, mb, n), jnp.float32),
            pltpu.SemaphoreType.DMA((NB,)),
            pltpu.SemaphoreType.DMA((NB,)),
            pltpu.SemaphoreType.DMA((NB,)),
            pltpu.SemaphoreType.DMA((NB,)),
        ],
        compiler_params=pltpu.CompilerParams(collective_id=0),
    )(x, W1, W2)


# device time: 20224 ns/iter; 2.4253x vs baseline; 1.3011x over previous
import jax
import jax.numpy as jnp
from jax import lax
from jax.experimental import pallas as pl
from jax.experimental.pallas import tpu as pltpu

N_DEV = 4
NB = 4


def kernel(x, W1, W2):
    m, k = x.shape
    _, h_per = W1.shape
    _, n = W2.shape
    mb = m // NB

    def body(x_ref, w1_ref, w2_ref, out_ref,
             xb_ref, w1b_ref, w2b_ref,
             accA, accB, sendA, sendB, recvA, recvB,
             sendA_sems, recvA_sems, sendB_sems, recvB_sems):
        p = lax.axis_index("i")
        nbrA = p ^ 1
        nbrB = 3 - p

        barrier_sem = pltpu.get_barrier_semaphore()
        for nbr in [nbrA, nbrB]:
            pl.semaphore_signal(
                barrier_sem, inc=1,
                device_id=(nbr,), device_id_type=pl.DeviceIdType.MESH,
            )
        pl.semaphore_wait(barrier_sem, 2)

        xb_ref[:, :] = x_ref[:, :].astype(jnp.bfloat16)
        w1b_ref[:, :] = w1_ref[:, :].astype(jnp.bfloat16)
        w2b_ref[:, :] = w2_ref[:, :].astype(jnp.bfloat16)

        rdmaA = [
            pltpu.make_async_remote_copy(
                src_ref=sendA.at[b], dst_ref=recvA.at[b],
                send_sem=sendA_sems.at[b], recv_sem=recvA_sems.at[b],
                device_id=(nbrA,), device_id_type=pl.DeviceIdType.MESH,
            )
            for b in range(NB)
        ]
        rdmaB = [
            pltpu.make_async_remote_copy(
                src_ref=sendB.at[b], dst_ref=recvB.at[b],
                send_sem=sendB_sems.at[b], recv_sem=recvB_sems.at[b],
                device_id=(nbrB,), device_id_type=pl.DeviceIdType.MESH,
            )
            for b in range(NB)
        ]

        for b in range(NB):
            hb = jnp.maximum(
                jnp.dot(
                    xb_ref[b * mb:(b + 1) * mb, :], w1b_ref[:, :],
                    preferred_element_type=jnp.float32,
                ),
                0.0,
            ).astype(jnp.bfloat16)
            pb = jnp.dot(hb, w2b_ref[:, :], preferred_element_type=jnp.float32)
            accA[b] = pb
            sendA[b] = pb.astype(jnp.bfloat16)
            rdmaA[b].start()

        for b in range(NB):
            rdmaA[b].wait_recv()
            pair = accA[b] + recvA[b].astype(jnp.float32)
            accB[b] = pair
            sendB[b] = pair.astype(jnp.bfloat16)
            rdmaB[b].start()
            if b >= 1:
                rdmaB[b - 1].wait_recv()
                out_ref[(b - 1) * mb:b * mb, :] = (
                    accB[b - 1] + recvB[b - 1].astype(jnp.float32)
                )
        rdmaB[NB - 1].wait_recv()
        out_ref[(NB - 1) * mb:, :] = (
            accB[NB - 1] + recvB[NB - 1].astype(jnp.float32)
        )

        for b in range(NB):
            rdmaA[b].wait_send()
            rdmaB[b].wait_send()

    return pl.pallas_call(
        body,
        out_shape=jax.ShapeDtypeStruct((m, n), jnp.float32),
        in_specs=[
            pl.BlockSpec(memory_space=pltpu.VMEM),
            pl.BlockSpec(memory_space=pltpu.VMEM),
            pl.BlockSpec(memory_space=pltpu.VMEM),
        ],
        out_specs=pl.BlockSpec(memory_space=pltpu.VMEM),
        scratch_shapes=[
            pltpu.VMEM((m, k), jnp.bfloat16),
            pltpu.VMEM((k, h_per), jnp.bfloat16),
            pltpu.VMEM((h_per, n), jnp.bfloat16),
            pltpu.VMEM((NB, mb, n), jnp.float32),
            pltpu.VMEM((NB, mb, n), jnp.float32),
            pltpu.VMEM((NB, mb, n), jnp.bfloat16),
            pltpu.VMEM((NB, mb, n), jnp.bfloat16),
            pltpu.VMEM((NB, mb, n), jnp.bfloat16),
            pltpu.VMEM((NB, mb, n), jnp.bfloat16),
            pltpu.SemaphoreType.DMA((NB,)),
            pltpu.SemaphoreType.DMA((NB,)),
            pltpu.SemaphoreType.DMA((NB,)),
            pltpu.SemaphoreType.DMA((NB,)),
        ],
        compiler_params=pltpu.CompilerParams(collective_id=0),
    )(x, W1, W2)


# device time: 19841 ns/iter; 2.4721x vs baseline; 1.0193x over previous
import jax
import jax.numpy as jnp
from jax import lax
from jax.experimental import pallas as pl
from jax.experimental.pallas import tpu as pltpu

N_DEV = 4
NB = 4


def kernel(x, W1, W2):
    m, k = x.shape
    _, h_per = W1.shape
    _, n = W2.shape
    mb = m // NB

    def body(x_ref, w1_ref, w2_ref, out_ref,
             xb_ref, w1b_ref, w2b_ref,
             accA, sendA, sendB, recvA, recvB,
             sendA_sems, recvA_sems, sendB_sems, recvB_sems):
        p = lax.axis_index("i")
        nbrA = p ^ 1
        nbrB = 3 - p

        barrier_sem = pltpu.get_barrier_semaphore()
        for nbr in [nbrA, nbrB]:
            pl.semaphore_signal(
                barrier_sem, inc=1,
                device_id=(nbr,), device_id_type=pl.DeviceIdType.MESH,
            )

        xb_ref[:, :] = x_ref[:, :].astype(jnp.bfloat16)
        w1b_ref[:, :] = w1_ref[:, :].astype(jnp.bfloat16)
        w2b_ref[:, :] = w2_ref[:, :].astype(jnp.bfloat16)

        rdmaA = [
            pltpu.make_async_remote_copy(
                src_ref=sendA.at[b], dst_ref=recvA.at[b],
                send_sem=sendA_sems.at[b], recv_sem=recvA_sems.at[b],
                device_id=(nbrA,), device_id_type=pl.DeviceIdType.MESH,
            )
            for b in range(NB)
        ]
        rdmaB = [
            pltpu.make_async_remote_copy(
                src_ref=sendB.at[b], dst_ref=recvB.at[b],
                send_sem=sendB_sems.at[b], recv_sem=recvB_sems.at[b],
                device_id=(nbrB,), device_id_type=pl.DeviceIdType.MESH,
            )
            for b in range(NB)
        ]

        for b in range(NB):
            hb = jnp.maximum(
                jnp.dot(
                    xb_ref[b * mb:(b + 1) * mb, :], w1b_ref[:, :],
                    preferred_element_type=jnp.float32,
                ),
                0.0,
            ).astype(jnp.bfloat16)
            pb = jnp.dot(hb, w2b_ref[:, :], preferred_element_type=jnp.float32)
            accA[b] = pb
            sendA[b] = pb.astype(jnp.bfloat16)
            if b == 0:
                pl.semaphore_wait(barrier_sem, 2)
            rdmaA[b].start()

        for b in range(NB):
            rdmaA[b].wait_recv()
            sendB[b] = (accA[b] + recvA[b].astype(jnp.float32)).astype(
                jnp.bfloat16
            )
            rdmaB[b].start()
            if b >= 1:
                rdmaB[b - 1].wait_recv()
                out_ref[(b - 1) * mb:b * mb, :] = (
                    accA[b - 1]
                    + recvA[b - 1].astype(jnp.float32)
                    + recvB[b - 1].astype(jnp.float32)
                )
        rdmaB[NB - 1].wait_recv()
        out_ref[(NB - 1) * mb:, :] = (
            accA[NB - 1]
            + recvA[NB - 1].astype(jnp.float32)
            + recvB[NB - 1].astype(jnp.float32)
        )

        for b in range(NB):
            rdmaA[b].wait_send()
            rdmaB[b].wait_send()

    return pl.pallas_call(
        body,
        out_shape=jax.ShapeDtypeStruct((m, n), jnp.float32),
        in_specs=[
            pl.BlockSpec(memory_space=pltpu.VMEM),
            pl.BlockSpec(memory_space=pltpu.VMEM),
            pl.BlockSpec(memory_space=pltpu.VMEM),
        ],
        out_specs=pl.BlockSpec(memory_space=pltpu.VMEM),
        scratch_shapes=[
            pltpu.VMEM((m, k), jnp.bfloat16),
            pltpu.VMEM((k, h_per), jnp.bfloat16),
            pltpu.VMEM((h_per, n), jnp.bfloat16),
            pltpu.VMEM((NB, mb, n), jnp.float32),
            pltpu.VMEM((NB, mb, n), jnp.bfloat16),
            pltpu.VMEM((NB, mb, n), jnp.bfloat16),
            pltpu.VMEM((NB, mb, n), jnp.bfloat16),
            pltpu.VMEM((NB, mb, n), jnp.bfloat16),
            pltpu.SemaphoreType.DMA((NB,)),
            pltpu.SemaphoreType.DMA((NB,)),
            pltpu.SemaphoreType.DMA((NB,)),
            pltpu.SemaphoreType.DMA((NB,)),
        ],
        compiler_params=pltpu.CompilerParams(collective_id=0),
    )(x, W1, W2)
